# baseline (speedup 1.0000x reference)
import functools

import jax
import jax.numpy as jnp
from jax import lax
from jax.experimental import pallas as pl
from jax.experimental.pallas import tpu as pltpu
from jax.experimental.pallas import tpu_sc as plsc

NC = 2
NS = 16
L = 16
NW = NC * NS

NVOX = 96 * 96 * 96
NPAIR = 16
TOTAL = NPAIR * NVOX

RB = 720
NBTC = 6
VTC = NBTC * RB * 128
PER_SC = (NVOX - VTC) // 2
CH = 13824
NCHUNK2 = PER_SC // (2 * CH)

NTHR = 32
NSEG = NTHR + 1
KQ = 4096
SENT = 33
NBANK = 8
BANKW = 1024
HTOT = NBANK * BANKW


def _body(inp_hbm, tar_hbm, s_hbm, lut_hbm, hist_out, sl1_out,
          s_v, lut_v, x0, x1, t0, t1, sx, st, hx, ht, sl1_v,
          sem_x0, sem_x1, sem_t0, sem_t1):
    wid = lax.axis_index("s") * NC + lax.axis_index("c")
    slab = lax.div(wid, jnp.int32(2))
    half = lax.rem(wid, jnp.int32(2))
    base = slab * NVOX + VTC + half * PER_SC

    pltpu.sync_copy(s_hbm, s_v)
    pltpu.sync_copy(lut_hbm, lut_v)

    lane = lax.iota(jnp.int32, L)
    ones = jnp.full((L,), 1.0, jnp.float32)
    zero_f = jnp.zeros((L,), jnp.float32)
    zero_i = jnp.zeros((L,), jnp.int32)

    def _zh(i, c):
        hx[pl.ds(i * L, L)] = zero_f
        ht[pl.ds(i * L, L)] = zero_f
        return c
    lax.fori_loop(0, HTOT // L, _zh, 0)

    def _zs(i, c):
        sx[pl.ds(i * L, L)] = zero_f
        st[pl.ds(i * L, L)] = zero_f
        return c
    lax.fori_loop(0, CH // L, _zs, 0)

    sA = s_v[pl.ds(0, L)]
    sB = s_v[pl.ds(L, L)]
    idx7 = jnp.full((L,), 7, jnp.int32)
    sA7 = jnp.take_along_axis(sA, idx7, axis=0, mode="promise_in_bounds")
    sB7 = jnp.take_along_axis(sB, idx7, axis=0, mode="promise_in_bounds")

    def rank16(tbl, t7, v):
        pos = jnp.where(t7 <= v, jnp.int32(8), jnp.int32(0))
        for step in (4, 2, 1):
            tv = jnp.take_along_axis(tbl, pos + jnp.int32(step - 1), axis=0,
                                     mode="promise_in_bounds")
            pos = pos + jnp.where(tv <= v, jnp.int32(step), jnp.int32(0))
        tv = jnp.take_along_axis(tbl, pos, axis=0, mode="promise_in_bounds")
        return pos + jnp.where(tv <= v, jnp.int32(1), jnp.int32(0))

    def rank(v):
        return rank16(sA, sA7, v) + rank16(sB, sB7, v)

    def process(xr, tr, carry):
        @plsc.parallel_loop(0, CH, L, unroll=8, carry=carry)
        def out(off, c):
            acc, ox, ot = c
            blane = ((off & jnp.int32(0x70)) << 6) + lane
            x = xr[pl.ds(off, L)]
            t = tr[pl.ds(off, L)]
            d = x - t
            ad = jnp.abs(d)
            acc = acc + jnp.where(ad < 1.0, (0.5 * d) * d, ad - 0.5)

            qx = (x * jnp.float32(KQ)).astype(jnp.int32)
            gx = plsc.load_gather(lut_v, [qx])
            plsc.addupdate_scatter(hx, [lax.shift_left(gx, 4) + blane], ones)
            fx = gx == jnp.int32(SENT)
            plsc.store_scatter(sx, [ox + lane], x)
            ox = ox + lax.shift_left(
                jnp.minimum(plsc.all_reduce_population_count(fx), jnp.int32(1)), 4)

            qt_ = (t * jnp.float32(KQ)).astype(jnp.int32)
            gt = plsc.load_gather(lut_v, [qt_])
            plsc.addupdate_scatter(ht, [lax.shift_left(gt, 4) + blane], ones)
            ft = gt == jnp.int32(SENT)
            plsc.store_scatter(st, [ot + lane], t)
            ot = ot + lax.shift_left(
                jnp.minimum(plsc.all_reduce_population_count(ft), jnp.int32(1)), 4)
            return acc, ox, ot
        return out

    def refine(side, h, off_v):
        nf = jnp.max(off_v)
        nit = lax.div(nf + jnp.int32(L - 1), jnp.int32(L))

        def b2(i, c):
            inb = (i * L + lane) < off_v
            v = side[pl.ds(i * L, L)]
            q = (v * jnp.float32(KQ)).astype(jnp.int32)
            g = plsc.load_gather(lut_v, [q])
            msk = inb & (g == jnp.int32(SENT))
            slot = lax.shift_left(rank(v), 4) + (((i & 7) << 10) + lane)
            plsc.addupdate_scatter(h, [slot], ones, mask=msk)
            return c
        lax.fori_loop(0, nit, b2, 0)

    def start0(g2):
        pltpu.async_copy(inp_hbm.at[pl.ds(base + g2 * (2 * CH), CH)], x0, sem_x0)
        pltpu.async_copy(tar_hbm.at[pl.ds(base + g2 * (2 * CH), CH)], t0, sem_t0)

    def wait0():
        pltpu.make_async_copy(inp_hbm.at[pl.ds(base, CH)], x0, sem_x0).wait()
        pltpu.make_async_copy(tar_hbm.at[pl.ds(base, CH)], t0, sem_t0).wait()

    start0(0)

    def pair_body(g2, acc):
        off1 = base + g2 * (2 * CH) + CH
        wait0()
        h1 = pltpu.async_copy(inp_hbm.at[pl.ds(off1, CH)], x1, sem_x1)
        h2 = pltpu.async_copy(tar_hbm.at[pl.ds(off1, CH)], t1, sem_t1)
        acc, ox, ot = process(x0, t0, (acc, zero_i, zero_i))
        refine(sx, hx, ox)
        refine(st, ht, ot)

        @pl.when(g2 < NCHUNK2 - 1)
        def _():
            start0(g2 + 1)

        h1.wait()
        h2.wait()
        acc, ox, ot = process(x1, t1, (acc, zero_i, zero_i))
        refine(sx, hx, ox)
        refine(st, ht, ot)
        return acc

    acc = lax.fori_loop(0, NCHUNK2, pair_body, zero_f)

    sl1_v[...] = acc
    pltpu.sync_copy(sl1_v, sl1_out.at[wid])
    pltpu.sync_copy(hx, hist_out.at[wid, 0])
    pltpu.sync_copy(ht, hist_out.at[wid, 1])


def _tc_body(bins_ref, x_ref, t_ref, out_ref):
    b = pl.program_id(1)

    @pl.when(b == 0)
    def _():
        out_ref[...] = jnp.zeros_like(out_ref)

    x = x_ref[0]
    t = t_ref[0]
    d = x - t
    ad = jnp.abs(d)
    sl1 = jnp.where(ad < 1.0, (0.5 * d) * d, ad - 0.5)
    ones_row = jnp.ones((1, RB), jnp.float32)

    def red(m):
        return jax.lax.dot_general(
            ones_row, m, (((1,), (0,)), ((), ())),
            preferred_element_type=jnp.float32)[0]

    out_ref[0, 32, :] += red(sl1)
    for j in range(16):
        lo = bins_ref[j, 0]
        hi = bins_ref[j, 1]
        out_ref[0, j, :] += red(jnp.where((x >= lo) & (x < hi), 1.0, 0.0))
        out_ref[0, 16 + j, :] += red(jnp.where((t >= lo) & (t < hi), 1.0, 0.0))


@jax.jit
def kernel(inp, tar, bin_range):
    inp_f = inp.reshape(TOTAL)
    tar_f = tar.reshape(TOTAL)
    s = jnp.sort(bin_range.reshape(NTHR))

    qt = (s * jnp.float32(KQ)).astype(jnp.int32)
    buckets = jnp.arange(KQ, dtype=jnp.int32)
    r0 = jnp.searchsorted(qt, buckets, side="left").astype(jnp.int32)
    isb = jnp.any(buckets[:, None] == qt[None, :], axis=1)
    lut = jnp.where(isb, jnp.int32(SENT), r0)

    mesh = plsc.VectorSubcoreMesh(core_axis_name="c", subcore_axis_name="s")
    hist, sl1 = pl.kernel(
        _body,
        mesh=mesh,
        compiler_params=pltpu.CompilerParams(needs_layout_passes=False),
        out_type=[
            jax.ShapeDtypeStruct((NW, 2, HTOT), jnp.float32),
            jax.ShapeDtypeStruct((NW, L), jnp.float32),
        ],
        scratch_types=[
            pltpu.VMEM((NTHR,), jnp.float32),
            pltpu.VMEM((KQ,), jnp.int32),
            pltpu.VMEM((CH,), jnp.float32),
            pltpu.VMEM((CH,), jnp.float32),
            pltpu.VMEM((CH,), jnp.float32),
            pltpu.VMEM((CH,), jnp.float32),
            pltpu.VMEM((CH,), jnp.float32),
            pltpu.VMEM((CH,), jnp.float32),
            pltpu.VMEM((HTOT,), jnp.float32),
            pltpu.VMEM((HTOT,), jnp.float32),
            pltpu.VMEM((L,), jnp.float32),
            pltpu.SemaphoreType.DMA,
            pltpu.SemaphoreType.DMA,
            pltpu.SemaphoreType.DMA,
            pltpu.SemaphoreType.DMA,
        ],
    )(inp_f, tar_f, s, lut)

    tc = pl.pallas_call(
        _tc_body,
        grid=(NPAIR, NBTC),
        in_specs=[
            pl.BlockSpec(memory_space=pltpu.SMEM),
            pl.BlockSpec((1, RB, 128), lambda i, b: (i, b, 0)),
            pl.BlockSpec((1, RB, 128), lambda i, b: (i, b, 0)),
        ],
        out_specs=pl.BlockSpec((1, 40, 128), lambda i, b: (i, 0, 0)),
        out_shape=jax.ShapeDtypeStruct((NPAIR, 40, 128), jnp.float32),
    )(bin_range, inp_f.reshape(NPAIR, NVOX // 128, 128),
      tar_f.reshape(NPAIR, NVOX // 128, 128))

    h = hist.reshape(NW, 2, NBANK, BANKW // L, L)[:, :, :, :NSEG, :]
    h = h.sum(axis=(2, 4))
    h = h.reshape(NPAIR, 2, 2, NSEG).sum(axis=1)
    cdf = jnp.cumsum(h, axis=-1)
    plo = jnp.searchsorted(s, bin_range[:, 0], side="left")
    phi = jnp.searchsorted(s, bin_range[:, 1], side="left")
    cnt = jnp.maximum(cdf[:, :, phi] - cdf[:, :, plo], 0.0)
    cnt_tc = tc[:, :32, :].sum(-1).reshape(NPAIR, 2, 16)
    freq = (cnt + cnt_tc) / NVOX
    loss2 = jnp.mean(jnp.abs(freq[:, 0, :] - freq[:, 1, :]))
    loss1 = (sl1.sum() + tc[:, 32, :].sum()) / TOTAL
    return 0.5 * loss1 + 0.5 * loss2


# speedup vs baseline: 1.0864x; 1.0864x over previous
import functools

import jax
import jax.numpy as jnp
from jax import lax
from jax.experimental import pallas as pl
from jax.experimental.pallas import tpu as pltpu
from jax.experimental.pallas import tpu_sc as plsc

NC = 2
NS = 16
L = 16
NW = NC * NS

NVOX = 96 * 96 * 96
NPAIR = 16
TOTAL = NPAIR * NVOX

RB = 720
NBTC = 6
VTC = NBTC * RB * 128
PER_SC = (NVOX - VTC) // 2
CH = 13824
NCHUNK2 = PER_SC // (2 * CH)

NTHR = 32
NSEG = NTHR + 1
KQ = 4096
SENT = 33
NBANK = 8
BANKW = 1024
HTOT = NBANK * BANKW


def _body(inp_hbm, tar_hbm, s_hbm, lut_hbm, hist_out, sl1_out,
          s_v, lut_v, x0, x1, t0, t1, sx, st, hx, ht, sl1_v,
          sem_x0, sem_x1, sem_t0, sem_t1):
    wid = lax.axis_index("s") * NC + lax.axis_index("c")
    slab = lax.div(wid, jnp.int32(2))
    half = lax.rem(wid, jnp.int32(2))
    base = slab * NVOX + VTC + half * PER_SC

    pltpu.sync_copy(s_hbm, s_v)
    pltpu.sync_copy(lut_hbm, lut_v)

    lane = lax.iota(jnp.int32, L)
    ones = jnp.full((L,), 1.0, jnp.float32)
    zero_f = jnp.zeros((L,), jnp.float32)
    zero_i = jnp.zeros((L,), jnp.int32)

    def _zh(i, c):
        hx[pl.ds(i * L, L)] = zero_f
        ht[pl.ds(i * L, L)] = zero_f
        return c
    lax.fori_loop(0, HTOT // L, _zh, 0)

    def _zs(i, c):
        sx[pl.ds(i * L, L)] = zero_f
        st[pl.ds(i * L, L)] = zero_f
        return c
    lax.fori_loop(0, CH // L, _zs, 0)

    sA = s_v[pl.ds(0, L)]
    sB = s_v[pl.ds(L, L)]
    idx7 = jnp.full((L,), 7, jnp.int32)
    sA7 = jnp.take_along_axis(sA, idx7, axis=0, mode="promise_in_bounds")
    sB7 = jnp.take_along_axis(sB, idx7, axis=0, mode="promise_in_bounds")

    def rank16(tbl, t7, v):
        pos = jnp.where(t7 <= v, jnp.int32(8), jnp.int32(0))
        for step in (4, 2, 1):
            tv = jnp.take_along_axis(tbl, pos + jnp.int32(step - 1), axis=0,
                                     mode="promise_in_bounds")
            pos = pos + jnp.where(tv <= v, jnp.int32(step), jnp.int32(0))
        tv = jnp.take_along_axis(tbl, pos, axis=0, mode="promise_in_bounds")
        return pos + jnp.where(tv <= v, jnp.int32(1), jnp.int32(0))

    def rank(v):
        return rank16(sA, sA7, v) + rank16(sB, sB7, v)

    def process(xr, tr, carry):
        @plsc.parallel_loop(0, CH, L, unroll=8, carry=carry)
        def out(off, c):
            acc, ox, ot = c
            blane = ((off & jnp.int32(0x70)) << 6) + lane
            x = xr[pl.ds(off, L)]
            t = tr[pl.ds(off, L)]
            d = x - t
            ad = jnp.abs(d)
            acc = acc + jnp.where(ad < 1.0, (0.5 * d) * d, ad - 0.5)

            qx = (x * jnp.float32(KQ)).astype(jnp.int32)
            gx = plsc.load_gather(lut_v, [qx])
            plsc.addupdate_scatter(hx, [lax.shift_left(gx, 4) + blane], ones)
            fx = gx == jnp.int32(SENT)
            plsc.store_scatter(sx, [ox + lane], x)
            ox = ox + lax.shift_left(
                jnp.minimum(plsc.all_reduce_population_count(fx), jnp.int32(1)), 4)

            qt_ = (t * jnp.float32(KQ)).astype(jnp.int32)
            gt = plsc.load_gather(lut_v, [qt_])
            plsc.addupdate_scatter(ht, [lax.shift_left(gt, 4) + blane], ones)
            ft = gt == jnp.int32(SENT)
            plsc.store_scatter(st, [ot + lane], t)
            ot = ot + lax.shift_left(
                jnp.minimum(plsc.all_reduce_population_count(ft), jnp.int32(1)), 4)
            return acc, ox, ot
        return out

    def refine(side, h, off_v):
        nf = jnp.max(off_v)
        nit = lax.div(nf + jnp.int32(L - 1), jnp.int32(L))

        def b2(i, c):
            inb = (i * L + lane) < off_v
            v = side[pl.ds(i * L, L)]
            q = (v * jnp.float32(KQ)).astype(jnp.int32)
            g = plsc.load_gather(lut_v, [q])
            msk = inb & (g == jnp.int32(SENT))
            slot = lax.shift_left(rank(v), 4) + (((i & 7) << 10) + lane)
            plsc.addupdate_scatter(h, [slot], ones, mask=msk)
            return c
        lax.fori_loop(0, nit, b2, 0)

    def start0(g2):
        pltpu.async_copy(inp_hbm.at[pl.ds(base + g2 * (2 * CH), CH)], x0, sem_x0)
        pltpu.async_copy(tar_hbm.at[pl.ds(base + g2 * (2 * CH), CH)], t0, sem_t0)

    def wait0():
        pltpu.make_async_copy(inp_hbm.at[pl.ds(base, CH)], x0, sem_x0).wait()
        pltpu.make_async_copy(tar_hbm.at[pl.ds(base, CH)], t0, sem_t0).wait()

    start0(0)

    def pair_body(g2, acc):
        off1 = base + g2 * (2 * CH) + CH
        wait0()
        h1 = pltpu.async_copy(inp_hbm.at[pl.ds(off1, CH)], x1, sem_x1)
        h2 = pltpu.async_copy(tar_hbm.at[pl.ds(off1, CH)], t1, sem_t1)
        acc, ox, ot = process(x0, t0, (acc, zero_i, zero_i))
        refine(sx, hx, ox)
        refine(st, ht, ot)

        @pl.when(g2 < NCHUNK2 - 1)
        def _():
            start0(g2 + 1)

        h1.wait()
        h2.wait()
        acc, ox, ot = process(x1, t1, (acc, zero_i, zero_i))
        refine(sx, hx, ox)
        refine(st, ht, ot)
        return acc

    acc = lax.fori_loop(0, NCHUNK2, pair_body, zero_f)

    sl1_v[...] = acc
    pltpu.sync_copy(sl1_v, sl1_out.at[wid])
    pltpu.sync_copy(hx, hist_out.at[wid, 0])
    pltpu.sync_copy(ht, hist_out.at[wid, 1])


def _tc_body(bins_ref, x_ref, t_ref, out_ref):
    b = pl.program_id(1)

    @pl.when(b == 0)
    def _():
        out_ref[...] = jnp.zeros_like(out_ref)

    x = x_ref[0]
    t = t_ref[0]
    d = x - t
    ad = jnp.abs(d)
    out_ref[0, 32, :] += jnp.sum(
        jnp.where(ad < 1.0, (0.5 * d) * d, ad - 0.5), axis=0)
    for j in range(16):
        lo = bins_ref[j, 0]
        hi = bins_ref[j, 1]
        out_ref[0, j, :] += jnp.sum(
            jnp.where((x >= lo) & (x < hi), 1.0, 0.0), axis=0)
        out_ref[0, 16 + j, :] += jnp.sum(
            jnp.where((t >= lo) & (t < hi), 1.0, 0.0), axis=0)


@jax.jit
def kernel(inp, tar, bin_range):
    inp_f = inp.reshape(TOTAL)
    tar_f = tar.reshape(TOTAL)
    s = jnp.sort(bin_range.reshape(NTHR))

    qt = (s * jnp.float32(KQ)).astype(jnp.int32)
    buckets = jnp.arange(KQ, dtype=jnp.int32)
    r0 = jnp.searchsorted(qt, buckets, side="left").astype(jnp.int32)
    isb = jnp.any(buckets[:, None] == qt[None, :], axis=1)
    lut = jnp.where(isb, jnp.int32(SENT), r0)

    mesh = plsc.VectorSubcoreMesh(core_axis_name="c", subcore_axis_name="s")
    hist, sl1 = pl.kernel(
        _body,
        mesh=mesh,
        compiler_params=pltpu.CompilerParams(needs_layout_passes=False),
        out_type=[
            jax.ShapeDtypeStruct((NW, 2, HTOT), jnp.float32),
            jax.ShapeDtypeStruct((NW, L), jnp.float32),
        ],
        scratch_types=[
            pltpu.VMEM((NTHR,), jnp.float32),
            pltpu.VMEM((KQ,), jnp.int32),
            pltpu.VMEM((CH,), jnp.float32),
            pltpu.VMEM((CH,), jnp.float32),
            pltpu.VMEM((CH,), jnp.float32),
            pltpu.VMEM((CH,), jnp.float32),
            pltpu.VMEM((CH,), jnp.float32),
            pltpu.VMEM((CH,), jnp.float32),
            pltpu.VMEM((HTOT,), jnp.float32),
            pltpu.VMEM((HTOT,), jnp.float32),
            pltpu.VMEM((L,), jnp.float32),
            pltpu.SemaphoreType.DMA,
            pltpu.SemaphoreType.DMA,
            pltpu.SemaphoreType.DMA,
            pltpu.SemaphoreType.DMA,
        ],
    )(inp_f, tar_f, s, lut)

    tc = pl.pallas_call(
        _tc_body,
        grid=(NPAIR, NBTC),
        in_specs=[
            pl.BlockSpec(memory_space=pltpu.SMEM),
            pl.BlockSpec((1, RB, 128), lambda i, b: (i, b, 0)),
            pl.BlockSpec((1, RB, 128), lambda i, b: (i, b, 0)),
        ],
        out_specs=pl.BlockSpec((1, 40, 128), lambda i, b: (i, 0, 0)),
        out_shape=jax.ShapeDtypeStruct((NPAIR, 40, 128), jnp.float32),
    )(bin_range, inp_f.reshape(NPAIR, NVOX // 128, 128),
      tar_f.reshape(NPAIR, NVOX // 128, 128))

    h = hist.reshape(NW, 2, NBANK, BANKW // L, L)[:, :, :, :NSEG, :]
    h = h.sum(axis=(2, 4))
    h = h.reshape(NPAIR, 2, 2, NSEG).sum(axis=1)
    cdf = jnp.cumsum(h, axis=-1)
    plo = jnp.searchsorted(s, bin_range[:, 0], side="left")
    phi = jnp.searchsorted(s, bin_range[:, 1], side="left")
    cnt = jnp.maximum(cdf[:, :, phi] - cdf[:, :, plo], 0.0)
    cnt_tc = tc[:, :32, :].sum(-1).reshape(NPAIR, 2, 16)
    freq = (cnt + cnt_tc) / NVOX
    loss2 = jnp.mean(jnp.abs(freq[:, 0, :] - freq[:, 1, :]))
    loss1 = (sl1.sum() + tc[:, 32, :].sum()) / TOTAL
    return 0.5 * loss1 + 0.5 * loss2
